# Initial kernel scaffold; baseline (speedup 1.0000x reference)
import jax
import jax.numpy as jnp
from jax import lax
from jax.experimental import pallas as pl
from jax.experimental.pallas import tpu as pltpu
from jax.experimental.pallas import tpu_sc as plsc

M = 2097152
N = 16384
NW = 32
RPW = N // NW
CH = 2048
KPC = CH // 16
NCHUNK = M // CH
CU_PAD = N + 8
T_THRESH = 1e-4
BG = 1.0

_I16 = lambda: lax.iota(jnp.int32, 16)


def _splat_i(x):
    return jnp.full((16,), x, jnp.int32)


def _splat_f(x):
    return jnp.full((16,), x, jnp.float32)


def _sload(ref, i):
    base = (i // 16) * 16
    v = ref[pl.ds(base, 16)]
    lane = i - base
    sel = jnp.where(_I16() == lane, v, jnp.int32(-2147483648))
    return jnp.max(sel)


def _body(sig_hbm, rgb_hbm, ts_hbm, cu_hbm,
          w_hbm, ws_hbm, d_hbm, img_hbm,
          cu_ref, sig_ref, ts_ref, rgb_ref, w_ref,
          ws_ref, d_ref, img_ref):
    wid = lax.axis_index("s") * 2 + lax.axis_index("c")
    r0 = wid * RPW
    r1 = r0 + RPW

    pltpu.sync_copy(cu_hbm, cu_ref)

    S = _sload(cu_ref, r0)
    E = _sload(cu_ref, r1)
    jH = jnp.minimum(S // CH, NCHUNK - 1)
    jA = (S + CH - 1) // CH
    jB = (E + CH - 1) // CH
    jB2 = jnp.maximum(jB, jH + 1)

    idx = _I16()
    zc = _splat_i(0)
    oc = _splat_i(1)
    tc = _splat_i(2)

    def process_one(g, o16, st):
        (r, cu_r, cu_r1, carry, aW, aD, aR, aG, aB, w_acc) = st
        gi = idx + g
        m = (gi >= cu_r) & (gi < cu_r1)
        o = idx + o16
        sig = sig_ref[pl.ds(o16, 16)]
        tv = plsc.load_gather(ts_ref, [o, zc])
        dtv = plsc.load_gather(ts_ref, [o, oc])
        tau = jnp.where(m, sig * dtv, 0.0)
        inc = plsc.cumsum(tau)
        excl = inc - tau
        tot = jnp.max(inc)
        T = jnp.exp(-(excl + carry))
        a = 1.0 - jnp.exp(-tau)
        w_r = jnp.where(T >= T_THRESH, a * T, 0.0)
        w_acc = jnp.where(m, w_r, w_acc)
        rv = plsc.load_gather(rgb_ref, [o, zc])
        gv = plsc.load_gather(rgb_ref, [o, oc])
        bv = plsc.load_gather(rgb_ref, [o, tc])
        aW = aW + w_r
        aD = aD + w_r * tv
        aR = aR + w_r * rv
        aG = aG + w_r * gv
        aB = aB + w_r * bv
        ends = cu_r1 <= g + 16
        downer = ends & (r < r1)

        def fin(_):
            rl = jnp.clip(r - r0, 0, RPW - 1)
            smask = (idx == 0) & downer
            plsc.store_scatter(ws_ref, [_splat_i(rl)], _splat_f(jnp.sum(aW)),
                               mask=smask)
            plsc.store_scatter(d_ref, [_splat_i(rl)], _splat_f(jnp.sum(aD)),
                               mask=smask)
            rgbv = jnp.where(idx == 0, jnp.sum(aR),
                             jnp.where(idx == 1, jnp.sum(aG), jnp.sum(aB)))
            plsc.store_scatter(img_ref, [_splat_i(rl), idx], rgbv,
                               mask=(idx < 3) & downer)
            return 0

        lax.cond(downer, fin, lambda _: 0, 0)

        r_n = jnp.where(ends, r + 1, r)
        carry_n = jnp.where(ends, 0.0, carry + tot)
        zf = jnp.float32(0.0)
        aW = jnp.where(ends, zf, aW)
        aD = jnp.where(ends, zf, aD)
        aR = jnp.where(ends, zf, aR)
        aG = jnp.where(ends, zf, aG)
        aB = jnp.where(ends, zf, aB)
        cu_r_n = jnp.where(ends, cu_r1, cu_r)
        nxt = _sload(cu_ref, jnp.minimum(r + 2, N))
        cu_r1_n = jnp.where(ends, nxt, cu_r1)
        return (r_n, cu_r_n, cu_r1_n, carry_n, aW, aD, aR, aG, aB, w_acc), ends

    def chunk_body(j, st):
        off = pl.multiple_of(j * CH, CH)
        pltpu.sync_copy(sig_hbm.at[pl.ds(off, CH)], sig_ref)
        pltpu.sync_copy(ts_hbm.at[pl.ds(off, CH)], ts_ref)
        pltpu.sync_copy(rgb_hbm.at[pl.ds(off, CH)], rgb_ref)

        def vreg_body(k, st):
            o16 = k * 16
            g = off + o16
            st = st[:9] + (_splat_f(0.0),)
            st, ends = process_one(g, o16, st)

            def wcond(c):
                s, e = c
                return e & (s[0] < N)

            def wbody(c):
                s, _ = c
                return process_one(g, o16, s)

            st, _ = lax.while_loop(wcond, wbody, (st, ends))
            w_ref[pl.ds(o16, 16)] = st[9]
            return st

        st = lax.fori_loop(0, KPC, vreg_body, st)

        @pl.when((j >= jA) & (j < jB))
        def _():
            pltpu.sync_copy(w_ref, w_hbm.at[pl.ds(off, CH)])

        return st

    st0 = (r0, S, _sload(cu_ref, r0 + 1), jnp.float32(0.0),
           _splat_f(0.0), _splat_f(0.0), _splat_f(0.0), _splat_f(0.0),
           _splat_f(0.0), _splat_f(0.0))
    lax.fori_loop(jH, jB2, chunk_body, st0)

    def blend_body(q, _):
        qb = q * 16
        flat = idx + qb
        row = flat // 3
        col = flat - row * 3
        v = plsc.load_gather(img_ref, [row, col])
        wsv = plsc.load_gather(ws_ref, [row])
        plsc.store_scatter(img_ref, [row, col], v + (1.0 - wsv) * BG)
        return 0

    lax.fori_loop(0, RPW * 3 // 16, blend_body, 0)

    pltpu.sync_copy(ws_ref, ws_hbm.at[pl.ds(r0, RPW)])
    pltpu.sync_copy(d_ref, d_hbm.at[pl.ds(r0, RPW)])
    pltpu.sync_copy(img_ref, img_hbm.at[pl.ds(r0, RPW)])


@jax.jit
def kernel(sigmas, rgbs, ts, cu_seqlens):
    cu_pad = jnp.concatenate(
        [cu_seqlens, jnp.full((CU_PAD - N - 1,), M, jnp.int32)])
    mesh = plsc.VectorSubcoreMesh(core_axis_name="c", subcore_axis_name="s")
    f = pl.kernel(
        _body,
        out_type=(
            jax.ShapeDtypeStruct((M,), jnp.float32),
            jax.ShapeDtypeStruct((N,), jnp.float32),
            jax.ShapeDtypeStruct((N,), jnp.float32),
            jax.ShapeDtypeStruct((N, 3), jnp.float32),
        ),
        mesh=mesh,
        scratch_types=[
            pltpu.VMEM((CU_PAD,), jnp.int32),
            pltpu.VMEM((CH,), jnp.float32),
            pltpu.VMEM((CH, 2), jnp.float32),
            pltpu.VMEM((CH, 3), jnp.float32),
            pltpu.VMEM((CH,), jnp.float32),
            pltpu.VMEM((RPW,), jnp.float32),
            pltpu.VMEM((RPW,), jnp.float32),
            pltpu.VMEM((RPW, 3), jnp.float32),
        ],
    )
    return f(sigmas, rgbs, ts, cu_pad)


# speedup vs baseline: 50.6153x; 50.6153x over previous
import jax
import jax.numpy as jnp
from jax import lax
from jax.experimental import pallas as pl
from jax.experimental.pallas import tpu as pltpu
from jax.experimental.pallas import tpu_sc as plsc

M = 2097152
N = 16384
NW = 32
RPW = N // NW
CH = 2048
KPC = CH // 16
NCHUNK = M // CH
CU_PAD = N + 8
T_THRESH = 1e-4
BG = 1.0

_I16 = lambda: lax.iota(jnp.int32, 16)


def _splat_i(x):
    return jnp.full((16,), x, jnp.int32)


def _splat_f(x):
    return jnp.full((16,), x, jnp.float32)


def _sload(ref, i):
    v = plsc.load_gather(ref, [_splat_i(i)])
    return v[0]


def _body(sig_hbm, rgb_hbm, ts_hbm, cu_hbm,
          w_hbm, ws_hbm, d_hbm, img_hbm,
          cu_ref, sig_ref, ts_ref, rgb_ref, w_ref,
          ws_ref, d_ref, img_ref):
    wid = lax.axis_index("s") * 2 + lax.axis_index("c")
    r0 = wid * RPW
    r1 = r0 + RPW

    pltpu.sync_copy(cu_hbm, cu_ref)

    S = _sload(cu_ref, r0)
    E = _sload(cu_ref, r1)
    jH = jnp.minimum(S // CH, NCHUNK - 1)
    jA = (S + CH - 1) // CH
    jB = (E + CH - 1) // CH
    jB2 = jnp.maximum(jB, jH + 1)

    idx = _I16()
    zc = _splat_i(0)
    oc = _splat_i(1)
    tc = _splat_i(2)

    def process_one(g, o16, st):
        (r, cu_r, cu_r1, carry, aW, aD, aR, aG, aB, w_acc) = st
        gi = idx + g
        m = (gi >= cu_r) & (gi < cu_r1)
        o = idx + o16
        sig = sig_ref[pl.ds(o16, 16)]
        tv = plsc.load_gather(ts_ref, [o, zc])
        dtv = plsc.load_gather(ts_ref, [o, oc])
        tau = jnp.where(m, sig * dtv, 0.0)
        inc = plsc.cumsum(tau)
        excl = inc - tau
        tot = jnp.max(inc)
        T = jnp.exp(-(excl + carry))
        a = 1.0 - jnp.exp(-tau)
        w_r = jnp.where(T >= T_THRESH, a * T, 0.0)
        w_acc = jnp.where(m, w_r, w_acc)
        rv = plsc.load_gather(rgb_ref, [o, zc])
        gv = plsc.load_gather(rgb_ref, [o, oc])
        bv = plsc.load_gather(rgb_ref, [o, tc])
        aW = aW + w_r
        aD = aD + w_r * tv
        aR = aR + w_r * rv
        aG = aG + w_r * gv
        aB = aB + w_r * bv
        ends = cu_r1 <= g + 16
        downer = ends & (r < r1)

        def fin(_):
            rl = jnp.clip(r - r0, 0, RPW - 1)
            smask = (idx == 0) & downer
            plsc.store_scatter(ws_ref, [_splat_i(rl)], _splat_f(jnp.sum(aW)),
                               mask=smask)
            plsc.store_scatter(d_ref, [_splat_i(rl)], _splat_f(jnp.sum(aD)),
                               mask=smask)
            rgbv = jnp.where(idx == 0, jnp.sum(aR),
                             jnp.where(idx == 1, jnp.sum(aG), jnp.sum(aB)))
            plsc.store_scatter(img_ref, [_splat_i(rl), idx], rgbv,
                               mask=(idx < 3) & downer)
            return 0

        lax.cond(downer, fin, lambda _: 0, 0)

        r_n = jnp.where(ends, r + 1, r)
        carry_n = jnp.where(ends, 0.0, carry + tot)
        zf = jnp.float32(0.0)
        aW = jnp.where(ends, zf, aW)
        aD = jnp.where(ends, zf, aD)
        aR = jnp.where(ends, zf, aR)
        aG = jnp.where(ends, zf, aG)
        aB = jnp.where(ends, zf, aB)
        cu_r_n = jnp.where(ends, cu_r1, cu_r)
        nxt = _sload(cu_ref, jnp.minimum(r + 2, N))
        cu_r1_n = jnp.where(ends, nxt, cu_r1)
        return (r_n, cu_r_n, cu_r1_n, carry_n, aW, aD, aR, aG, aB, w_acc), ends

    def chunk_body(j, st):
        off = pl.multiple_of(j * CH, CH)
        pltpu.sync_copy(sig_hbm.at[pl.ds(off, CH)], sig_ref)
        pltpu.sync_copy(ts_hbm.at[pl.ds(off, CH)], ts_ref)
        pltpu.sync_copy(rgb_hbm.at[pl.ds(off, CH)], rgb_ref)

        def vreg_body(k, st):
            o16 = k * 16
            g = off + o16
            st = st[:9] + (_splat_f(0.0),)
            st, ends = process_one(g, o16, st)

            def wcond(c):
                s, e = c
                return e & (s[0] < N)

            def wbody(c):
                s, _ = c
                return process_one(g, o16, s)

            st, _ = lax.while_loop(wcond, wbody, (st, ends))
            w_ref[pl.ds(o16, 16)] = st[9]
            return st

        st = lax.fori_loop(0, KPC, vreg_body, st)

        @pl.when((j >= jA) & (j < jB))
        def _():
            pltpu.sync_copy(w_ref, w_hbm.at[pl.ds(off, CH)])

        return st

    st0 = (r0, S, _sload(cu_ref, r0 + 1), jnp.float32(0.0),
           _splat_f(0.0), _splat_f(0.0), _splat_f(0.0), _splat_f(0.0),
           _splat_f(0.0), _splat_f(0.0))
    lax.fori_loop(jH, jB2, chunk_body, st0)

    def blend_body(q, _):
        qb = q * 16
        flat = idx + qb
        row = flat // 3
        col = flat - row * 3
        v = plsc.load_gather(img_ref, [row, col])
        wsv = plsc.load_gather(ws_ref, [row])
        plsc.store_scatter(img_ref, [row, col], v + (1.0 - wsv) * BG)
        return 0

    lax.fori_loop(0, RPW * 3 // 16, blend_body, 0)

    pltpu.sync_copy(ws_ref, ws_hbm.at[pl.ds(r0, RPW)])
    pltpu.sync_copy(d_ref, d_hbm.at[pl.ds(r0, RPW)])
    pltpu.sync_copy(img_ref, img_hbm.at[pl.ds(r0, RPW)])


@jax.jit
def kernel(sigmas, rgbs, ts, cu_seqlens):
    cu_pad = jnp.concatenate(
        [cu_seqlens, jnp.full((CU_PAD - N - 1,), M, jnp.int32)])
    mesh = plsc.VectorSubcoreMesh(core_axis_name="c", subcore_axis_name="s")
    f = pl.kernel(
        _body,
        out_type=(
            jax.ShapeDtypeStruct((M,), jnp.float32),
            jax.ShapeDtypeStruct((N,), jnp.float32),
            jax.ShapeDtypeStruct((N,), jnp.float32),
            jax.ShapeDtypeStruct((N, 3), jnp.float32),
        ),
        mesh=mesh,
        compiler_params=pltpu.CompilerParams(
            needs_layout_passes=False, use_tc_tiling_on_sc=False),
        scratch_types=[
            pltpu.VMEM((CU_PAD,), jnp.int32),
            pltpu.VMEM((CH,), jnp.float32),
            pltpu.VMEM((CH, 2), jnp.float32),
            pltpu.VMEM((CH, 3), jnp.float32),
            pltpu.VMEM((CH,), jnp.float32),
            pltpu.VMEM((RPW,), jnp.float32),
            pltpu.VMEM((RPW,), jnp.float32),
            pltpu.VMEM((RPW, 3), jnp.float32),
        ],
    )
    return f(sigmas, rgbs, ts, cu_pad)


# speedup vs baseline: 63.9081x; 1.2626x over previous
import jax
import jax.numpy as jnp
from jax import lax
from jax.experimental import pallas as pl
from jax.experimental.pallas import tpu as pltpu
from jax.experimental.pallas import tpu_sc as plsc

M = 2097152
N = 16384
NW = 32
RPW = N // NW
CH = 2048
KPC = CH // 16
NCHUNK = M // CH
CU_PAD = N + 8
T_THRESH = 1e-4
BG = 1.0

_I16 = lambda: lax.iota(jnp.int32, 16)


def _splat_i(x):
    return jnp.full((16,), x, jnp.int32)


def _splat_f(x):
    return jnp.full((16,), x, jnp.float32)


def _sload(ref, i):
    v = plsc.load_gather(ref, [_splat_i(i)])
    return v[0]


def _body(sig_hbm, rgb_hbm, ts_hbm, cu_hbm,
          w_hbm, ws_hbm, d_hbm, img_hbm,
          cu_ref, sig_ref, ts_ref, rgb_ref, w_ref,
          ws_ref, d_ref, img_ref, sems, wsem):
    wid = lax.axis_index("s") * 2 + lax.axis_index("c")
    r0 = wid * RPW
    r1 = r0 + RPW

    pltpu.sync_copy(cu_hbm, cu_ref)

    S = _sload(cu_ref, r0)
    E = _sload(cu_ref, r1)
    jH = jnp.minimum(S // CH, NCHUNK - 1)
    jA = (S + CH - 1) // CH
    jB = (E + CH - 1) // CH
    jB2 = jnp.maximum(jB, jH + 1)

    idx = _I16()

    def in_dma(j, slot):
        off = pl.multiple_of(j * CH, CH)
        return (
            pltpu.make_async_copy(sig_hbm.at[pl.ds(off, CH)],
                                  sig_ref.at[slot], sems.at[slot, 0]),
            pltpu.make_async_copy(ts_hbm.at[pl.ds(2 * off, 2 * CH)],
                                  ts_ref.at[slot], sems.at[slot, 1]),
            pltpu.make_async_copy(rgb_hbm.at[pl.ds(3 * off, 3 * CH)],
                                  rgb_ref.at[slot], sems.at[slot, 2]),
        )

    def start_in(j, slot):
        for c in in_dma(j, slot):
            c.start()

    def wait_in(j, slot):
        for c in in_dma(j, slot):
            c.wait()

    def process_one(g, o16, slot, st):
        (r, cu_r, cu_r1, carry, aW, aD, aR, aG, aB, w_acc) = st
        gi = idx + g
        m = (gi >= cu_r) & (gi < cu_r1)
        o2 = idx * 2 + (2 * o16)
        o3 = idx * 3 + (3 * o16)
        sig = sig_ref[slot, pl.ds(o16, 16)]
        tv = plsc.load_gather(ts_ref, [_splat_i(slot), o2])
        dtv = plsc.load_gather(ts_ref, [_splat_i(slot), o2 + 1])
        tau = jnp.where(m, sig * dtv, 0.0)
        inc = plsc.cumsum(tau)
        excl = inc - tau
        tot = inc[15]
        T = jnp.exp(-(excl + carry))
        a = 1.0 - jnp.exp(-tau)
        w_r = jnp.where(T >= T_THRESH, a * T, 0.0)
        w_acc = jnp.where(m, w_r, w_acc)
        rv = plsc.load_gather(rgb_ref, [_splat_i(slot), o3])
        gv = plsc.load_gather(rgb_ref, [_splat_i(slot), o3 + 1])
        bv = plsc.load_gather(rgb_ref, [_splat_i(slot), o3 + 2])
        aW = aW + w_r
        aD = aD + w_r * tv
        aR = aR + w_r * rv
        aG = aG + w_r * gv
        aB = aB + w_r * bv
        ends = cu_r1 <= g + 16

        def slow(_):
            downer = r < r1
            rl = jnp.clip(r - r0, 0, RPW - 1)
            smask = (idx == 0) & downer
            plsc.store_scatter(ws_ref, [_splat_i(rl)], _splat_f(jnp.sum(aW)),
                               mask=smask)
            plsc.store_scatter(d_ref, [_splat_i(rl)], _splat_f(jnp.sum(aD)),
                               mask=smask)
            rgbv = jnp.where(idx == 0, jnp.sum(aR),
                             jnp.where(idx == 1, jnp.sum(aG), jnp.sum(aB)))
            plsc.store_scatter(img_ref, [_splat_i(rl), idx], rgbv,
                               mask=(idx < 3) & downer)
            nxt = _sload(cu_ref, jnp.minimum(r + 2, N))
            zv = _splat_f(0.0)
            return (r + 1, cu_r1, nxt, jnp.float32(0.0),
                    zv, zv, zv, zv, zv, w_acc)

        def fast(_):
            return (r, cu_r, cu_r1, carry + tot,
                    aW, aD, aR, aG, aB, w_acc)

        st = lax.cond(ends, slow, fast, 0)
        return st, ends

    def chunk_body(j, st):
        slot = (j - jH) % 2
        nslot = 1 - slot

        @pl.when(j + 1 < jB2)
        def _():
            start_in(j + 1, nslot)

        wait_in(j, slot)
        off = pl.multiple_of(j * CH, CH)

        @pl.when((j - jH >= 2) & (jnp.maximum(j - 2, 0) >= jA))
        def _():
            offp = pl.multiple_of(jnp.maximum(j - 2, 0) * CH, CH)
            pltpu.make_async_copy(w_ref.at[slot],
                                  w_hbm.at[pl.ds(offp, CH)],
                                  wsem.at[slot]).wait()

        def vreg_body(k, st):
            o16 = k * 16
            g = off + o16
            st = st[:9] + (_splat_f(0.0),)
            st, ends = process_one(g, o16, slot, st)

            def wcond(c):
                s, e = c
                return e & (s[0] < N)

            def wbody(c):
                s, _ = c
                return process_one(g, o16, slot, s)

            st, _ = lax.while_loop(wcond, wbody, (st, ends))
            w_ref[slot, pl.ds(o16, 16)] = st[9]
            return st

        st = lax.fori_loop(0, KPC, vreg_body, st)

        @pl.when((j >= jA) & (j < jB))
        def _():
            pltpu.make_async_copy(w_ref.at[slot],
                                  w_hbm.at[pl.ds(off, CH)],
                                  wsem.at[slot]).start()

        return st

    st0 = (r0, S, _sload(cu_ref, r0 + 1), jnp.float32(0.0),
           _splat_f(0.0), _splat_f(0.0), _splat_f(0.0), _splat_f(0.0),
           _splat_f(0.0), _splat_f(0.0))
    start_in(jH, 0)
    lax.fori_loop(jH, jB2, chunk_body, st0)

    def drain(jj):
        @pl.when((jj >= jH) & (jj >= jA) & (jj < jB))
        def _():
            offp = pl.multiple_of(jnp.maximum(jj, 0) * CH, CH)
            slotp = (jj - jH) % 2
            pltpu.make_async_copy(w_ref.at[slotp],
                                  w_hbm.at[pl.ds(offp, CH)],
                                  wsem.at[slotp]).wait()

    drain(jB2 - 2)
    drain(jB2 - 1)

    def blend_body(q, _):
        qb = q * 16
        flat = idx + qb
        row = flat // 3
        col = flat - row * 3
        v = plsc.load_gather(img_ref, [row, col])
        wsv = plsc.load_gather(ws_ref, [row])
        plsc.store_scatter(img_ref, [row, col], v + (1.0 - wsv) * BG)
        return 0

    lax.fori_loop(0, RPW * 3 // 16, blend_body, 0)

    pltpu.sync_copy(ws_ref, ws_hbm.at[pl.ds(r0, RPW)])
    pltpu.sync_copy(d_ref, d_hbm.at[pl.ds(r0, RPW)])
    pltpu.sync_copy(img_ref, img_hbm.at[pl.ds(r0, RPW)])


@jax.jit
def kernel(sigmas, rgbs, ts, cu_seqlens):
    cu_pad = jnp.concatenate(
        [cu_seqlens, jnp.full((CU_PAD - N - 1,), M, jnp.int32)])
    mesh = plsc.VectorSubcoreMesh(core_axis_name="c", subcore_axis_name="s")
    f = pl.kernel(
        _body,
        out_type=(
            jax.ShapeDtypeStruct((M,), jnp.float32),
            jax.ShapeDtypeStruct((N,), jnp.float32),
            jax.ShapeDtypeStruct((N,), jnp.float32),
            jax.ShapeDtypeStruct((N, 3), jnp.float32),
        ),
        mesh=mesh,
        compiler_params=pltpu.CompilerParams(
            needs_layout_passes=False, use_tc_tiling_on_sc=False),
        scratch_types=[
            pltpu.VMEM((CU_PAD,), jnp.int32),
            pltpu.VMEM((2, CH), jnp.float32),
            pltpu.VMEM((2, 2 * CH), jnp.float32),
            pltpu.VMEM((2, 3 * CH), jnp.float32),
            pltpu.VMEM((2, CH), jnp.float32),
            pltpu.VMEM((RPW,), jnp.float32),
            pltpu.VMEM((RPW,), jnp.float32),
            pltpu.VMEM((RPW, 3), jnp.float32),
            pltpu.SemaphoreType.DMA((2, 3)),
            pltpu.SemaphoreType.DMA((2,)),
        ],
    )
    return f(sigmas, rgbs.reshape(-1), ts.reshape(-1), cu_pad)


# speedup vs baseline: 249.2973x; 3.9009x over previous
import jax
import jax.numpy as jnp
from jax import lax
from jax.experimental import pallas as pl
from jax.experimental.pallas import tpu as pltpu
from jax.experimental.pallas import tpu_sc as plsc

M = 2097152
N = 16384
NW = 32
RPW = N // NW
CH = 2048
KPC = CH // 16
NCHUNK = M // CH
CU_PAD = N + 8
T_THRESH = 1e-4
BG = 1.0

_I16 = lambda: lax.iota(jnp.int32, 16)


def _splat_i(x):
    return jnp.full((16,), x, jnp.int32)


def _splat_f(x):
    return jnp.full((16,), x, jnp.float32)


def _sload(ref, i):
    v = plsc.load_gather(ref, [_splat_i(i)])
    return v[0]


def _body(vals_hbm, cu_hbm,
          w_hbm, ws_hbm, d_hbm, img_hbm,
          cu_ref, vals_ref, w_ref,
          ws_ref, d_ref, img_ref, sems, wsem):
    wid = lax.axis_index("s") * 2 + lax.axis_index("c")
    r0 = wid * RPW
    r1 = r0 + RPW

    pltpu.sync_copy(cu_hbm, cu_ref)

    S = _sload(cu_ref, r0)
    E = _sload(cu_ref, r1)
    jH = jnp.minimum(S // CH, NCHUNK - 1)
    jA = (S + CH - 1) // CH
    jB = (E + CH - 1) // CH
    jB2 = jnp.maximum(jB, jH + 1)

    idx = _I16()

    def in_dma(j, slot):
        off = pl.multiple_of(j * CH, CH)
        return pltpu.make_async_copy(vals_hbm.at[:, pl.ds(off, CH)],
                                     vals_ref.at[slot], sems.at[slot])

    def process_one(g, o16, slot, st):
        (r, cu_r, cu_r1, carry, aW, aD, aR, aG, aB, w_acc) = st
        gi = idx + g
        m = (gi >= cu_r) & (gi < cu_r1)
        sig = vals_ref[slot, 0, pl.ds(o16, 16)]
        tv = vals_ref[slot, 1, pl.ds(o16, 16)]
        dtv = vals_ref[slot, 2, pl.ds(o16, 16)]
        tau = jnp.where(m, sig * dtv, 0.0)
        inc = plsc.cumsum(tau)
        excl = inc - tau
        tot = inc[15]
        T = jnp.exp(-(excl + carry))
        a = 1.0 - jnp.exp(-tau)
        w_r = jnp.where(T >= T_THRESH, a * T, 0.0)
        w_acc = jnp.where(m, w_r, w_acc)
        rv = vals_ref[slot, 3, pl.ds(o16, 16)]
        gv = vals_ref[slot, 4, pl.ds(o16, 16)]
        bv = vals_ref[slot, 5, pl.ds(o16, 16)]
        aW = aW + w_r
        aD = aD + w_r * tv
        aR = aR + w_r * rv
        aG = aG + w_r * gv
        aB = aB + w_r * bv
        ends = cu_r1 <= g + 16

        def slow(_):
            downer = r < r1
            rl = jnp.clip(r - r0, 0, RPW - 1)
            smask = (idx == 0) & downer
            plsc.store_scatter(ws_ref, [_splat_i(rl)], _splat_f(jnp.sum(aW)),
                               mask=smask)
            plsc.store_scatter(d_ref, [_splat_i(rl)], _splat_f(jnp.sum(aD)),
                               mask=smask)
            rgbv = jnp.where(idx == 0, jnp.sum(aR),
                             jnp.where(idx == 1, jnp.sum(aG), jnp.sum(aB)))
            plsc.store_scatter(img_ref, [_splat_i(rl), idx], rgbv,
                               mask=(idx < 3) & downer)
            nxt = _sload(cu_ref, jnp.minimum(r + 2, N))
            zv = _splat_f(0.0)
            return (r + 1, cu_r1, nxt, jnp.float32(0.0),
                    zv, zv, zv, zv, zv, w_acc)

        def fast(_):
            return (r, cu_r, cu_r1, carry + tot,
                    aW, aD, aR, aG, aB, w_acc)

        st = lax.cond(ends, slow, fast, 0)
        return st, ends

    def chunk_body(j, st):
        slot = (j - jH) % 2
        nslot = 1 - slot

        @pl.when(j + 1 < jB2)
        def _():
            in_dma(j + 1, nslot).start()

        in_dma(j, slot).wait()
        off = pl.multiple_of(j * CH, CH)

        @pl.when((j - jH >= 2) & (jnp.maximum(j - 2, 0) >= jA))
        def _():
            offp = pl.multiple_of(jnp.maximum(j - 2, 0) * CH, CH)
            pltpu.make_async_copy(w_ref.at[slot],
                                  w_hbm.at[pl.ds(offp, CH)],
                                  wsem.at[slot]).wait()

        def vreg_body(k, st):
            o16 = k * 16
            g = off + o16
            st = st[:9] + (_splat_f(0.0),)
            st, ends = process_one(g, o16, slot, st)

            def wcond(c):
                s, e = c
                return e & (s[0] < N)

            def wbody(c):
                s, _ = c
                return process_one(g, o16, slot, s)

            st, _ = lax.while_loop(wcond, wbody, (st, ends))
            w_ref[slot, pl.ds(o16, 16)] = st[9]
            return st

        st = lax.fori_loop(0, KPC, vreg_body, st)

        @pl.when((j >= jA) & (j < jB))
        def _():
            pltpu.make_async_copy(w_ref.at[slot],
                                  w_hbm.at[pl.ds(off, CH)],
                                  wsem.at[slot]).start()

        return st

    st0 = (r0, S, _sload(cu_ref, r0 + 1), jnp.float32(0.0),
           _splat_f(0.0), _splat_f(0.0), _splat_f(0.0), _splat_f(0.0),
           _splat_f(0.0), _splat_f(0.0))
    in_dma(jH, 0).start()
    lax.fori_loop(jH, jB2, chunk_body, st0)

    def drain(jj):
        @pl.when((jj >= jH) & (jj >= jA) & (jj < jB))
        def _():
            offp = pl.multiple_of(jnp.maximum(jj, 0) * CH, CH)
            slotp = (jj - jH) % 2
            pltpu.make_async_copy(w_ref.at[slotp],
                                  w_hbm.at[pl.ds(offp, CH)],
                                  wsem.at[slotp]).wait()

    drain(jB2 - 2)
    drain(jB2 - 1)

    def blend_body(q, _):
        qb = q * 16
        flat = idx + qb
        row = flat // 3
        col = flat - row * 3
        v = plsc.load_gather(img_ref, [row, col])
        wsv = plsc.load_gather(ws_ref, [row])
        plsc.store_scatter(img_ref, [row, col], v + (1.0 - wsv) * BG)
        return 0

    lax.fori_loop(0, RPW * 3 // 16, blend_body, 0)

    pltpu.sync_copy(ws_ref, ws_hbm.at[pl.ds(r0, RPW)])
    pltpu.sync_copy(d_ref, d_hbm.at[pl.ds(r0, RPW)])
    pltpu.sync_copy(img_ref, img_hbm.at[pl.ds(r0, RPW)])


@jax.jit
def kernel(sigmas, rgbs, ts, cu_seqlens):
    cu_pad = jnp.concatenate(
        [cu_seqlens, jnp.full((CU_PAD - N - 1,), M, jnp.int32)])
    vals = jnp.stack([sigmas, ts[:, 0], ts[:, 1],
                      rgbs[:, 0], rgbs[:, 1], rgbs[:, 2]])
    mesh = plsc.VectorSubcoreMesh(core_axis_name="c", subcore_axis_name="s")
    f = pl.kernel(
        _body,
        out_type=(
            jax.ShapeDtypeStruct((M,), jnp.float32),
            jax.ShapeDtypeStruct((N,), jnp.float32),
            jax.ShapeDtypeStruct((N,), jnp.float32),
            jax.ShapeDtypeStruct((N, 3), jnp.float32),
        ),
        mesh=mesh,
        compiler_params=pltpu.CompilerParams(
            needs_layout_passes=False, use_tc_tiling_on_sc=False),
        scratch_types=[
            pltpu.VMEM((CU_PAD,), jnp.int32),
            pltpu.VMEM((2, 6, CH), jnp.float32),
            pltpu.VMEM((2, CH), jnp.float32),
            pltpu.VMEM((RPW,), jnp.float32),
            pltpu.VMEM((RPW,), jnp.float32),
            pltpu.VMEM((RPW, 3), jnp.float32),
            pltpu.SemaphoreType.DMA((2,)),
            pltpu.SemaphoreType.DMA((2,)),
        ],
    )
    return f(vals, cu_pad)


# speedup vs baseline: 1094.1094x; 4.3888x over previous
import jax
import jax.numpy as jnp
from jax import lax
from jax.experimental import pallas as pl
from jax.experimental.pallas import tpu as pltpu
from jax.experimental.pallas import tpu_sc as plsc

M = 2097152
N = 16384
NW = 32
RPW = N // NW
CH = 2048
KPC = CH // 16
NCHUNK = M // CH
CU_PAD = N + 8
T_THRESH = 1e-4
BG = 1.0

_I16 = lambda: lax.iota(jnp.int32, 16)


def _splat_i(x):
    return jnp.full((16,), x, jnp.int32)


def _splat_f(x):
    return jnp.full((16,), x, jnp.float32)


def _sload(ref, i):
    v = plsc.load_gather(ref, [_splat_i(i)])
    return v[0]


def _body(sig_hbm, ts_hbm, rgb_hbm, cu_hbm,
          w_hbm, ws_hbm, d_hbm, img_hbm,
          cu_ref, sig_ref, ts_ref, rgb_ref, w_ref,
          ws_ref, d_ref, img_ref, sems, wsem):
    wid = lax.axis_index("s") * 2 + lax.axis_index("c")
    r0 = wid * RPW
    r1 = r0 + RPW

    pltpu.sync_copy(cu_hbm, cu_ref)

    S = _sload(cu_ref, r0)
    E = _sload(cu_ref, r1)
    jH = jnp.minimum(S // CH, NCHUNK - 1)
    jA = (S + CH - 1) // CH
    jB = (E + CH - 1) // CH
    jB2 = jnp.maximum(jB, jH + 1)

    idx = _I16()

    def in_dma(j, slot):
        off = pl.multiple_of(j * CH, CH)
        blk = pl.multiple_of(j * (CH // 128), CH // 128)
        return (
            pltpu.make_async_copy(sig_hbm.at[pl.ds(off, CH)],
                                  sig_ref.at[slot], sems.at[slot, 0]),
            pltpu.make_async_copy(ts_hbm.at[pl.ds(blk, CH // 128)],
                                  ts_ref.at[slot], sems.at[slot, 1]),
            pltpu.make_async_copy(rgb_hbm.at[pl.ds(blk, CH // 128)],
                                  rgb_ref.at[slot], sems.at[slot, 2]),
        )

    def start_in(j, slot):
        for c in in_dma(j, slot):
            c.start()

    def wait_in(j, slot):
        for c in in_dma(j, slot):
            c.wait()

    def process_one(g, o16, slot, st):
        (r, cu_r, cu_r1, carry, aW, aD, aR, aG, aB, w_acc) = st
        gi = idx + g
        m = (gi >= cu_r) & (gi < cu_r1)
        b = o16 // 128
        l = pl.multiple_of(o16 - b * 128, 16)
        sig = sig_ref[slot, pl.ds(o16, 16)]
        tv = ts_ref[slot, b, 0, pl.ds(l, 16)]
        dtv = ts_ref[slot, b, 1, pl.ds(l, 16)]
        tau = jnp.where(m, sig * dtv, 0.0)
        inc = plsc.cumsum(tau)
        excl = inc - tau
        tot = inc[15]
        T = jnp.exp(-(excl + carry))
        a = 1.0 - jnp.exp(-tau)
        w_r = jnp.where(T >= T_THRESH, a * T, 0.0)
        w_acc = jnp.where(m, w_r, w_acc)
        rv = rgb_ref[slot, b, 0, pl.ds(l, 16)]
        gv = rgb_ref[slot, b, 1, pl.ds(l, 16)]
        bv = rgb_ref[slot, b, 2, pl.ds(l, 16)]
        aW = aW + w_r
        aD = aD + w_r * tv
        aR = aR + w_r * rv
        aG = aG + w_r * gv
        aB = aB + w_r * bv
        ends = cu_r1 <= g + 16

        def slow(_):
            downer = r < r1
            rl = jnp.clip(r - r0, 0, RPW - 1)
            smask = (idx == 0) & downer
            plsc.store_scatter(ws_ref, [_splat_i(rl)], _splat_f(jnp.sum(aW)),
                               mask=smask)
            plsc.store_scatter(d_ref, [_splat_i(rl)], _splat_f(jnp.sum(aD)),
                               mask=smask)
            rgbv = jnp.where(idx == 0, jnp.sum(aR),
                             jnp.where(idx == 1, jnp.sum(aG), jnp.sum(aB)))
            plsc.store_scatter(img_ref, [_splat_i(rl), idx], rgbv,
                               mask=(idx < 3) & downer)
            nxt = _sload(cu_ref, jnp.minimum(r + 2, N))
            zv = _splat_f(0.0)
            return (r + 1, cu_r1, nxt, jnp.float32(0.0),
                    zv, zv, zv, zv, zv, w_acc)

        def fast(_):
            return (r, cu_r, cu_r1, carry + tot,
                    aW, aD, aR, aG, aB, w_acc)

        st = lax.cond(ends, slow, fast, 0)
        return st, ends

    def chunk_body(j, st):
        slot = (j - jH) % 2
        nslot = 1 - slot

        @pl.when(j + 1 < jB2)
        def _():
            start_in(j + 1, nslot)

        wait_in(j, slot)
        off = pl.multiple_of(j * CH, CH)

        @pl.when((j - jH >= 2) & (jnp.maximum(j - 2, 0) >= jA))
        def _():
            offp = pl.multiple_of(jnp.maximum(j - 2, 0) * CH, CH)
            pltpu.make_async_copy(w_ref.at[slot],
                                  w_hbm.at[pl.ds(offp, CH)],
                                  wsem.at[slot]).wait()

        def vreg_body(k, st):
            o16 = k * 16
            g = off + o16
            st = st[:9] + (_splat_f(0.0),)
            st, ends = process_one(g, o16, slot, st)

            def wcond(c):
                s, e = c
                return e & (s[0] < N)

            def wbody(c):
                s, _ = c
                return process_one(g, o16, slot, s)

            st, _ = lax.while_loop(wcond, wbody, (st, ends))
            w_ref[slot, pl.ds(o16, 16)] = st[9]
            return st

        st = lax.fori_loop(0, KPC, vreg_body, st)

        @pl.when((j >= jA) & (j < jB))
        def _():
            pltpu.make_async_copy(w_ref.at[slot],
                                  w_hbm.at[pl.ds(off, CH)],
                                  wsem.at[slot]).start()

        return st

    st0 = (r0, S, _sload(cu_ref, r0 + 1), jnp.float32(0.0),
           _splat_f(0.0), _splat_f(0.0), _splat_f(0.0), _splat_f(0.0),
           _splat_f(0.0), _splat_f(0.0))
    start_in(jH, 0)
    lax.fori_loop(jH, jB2, chunk_body, st0)

    def drain(jj):
        @pl.when((jj >= jH) & (jj >= jA) & (jj < jB))
        def _():
            offp = pl.multiple_of(jnp.maximum(jj, 0) * CH, CH)
            slotp = (jj - jH) % 2
            pltpu.make_async_copy(w_ref.at[slotp],
                                  w_hbm.at[pl.ds(offp, CH)],
                                  wsem.at[slotp]).wait()

    drain(jB2 - 2)
    drain(jB2 - 1)

    def blend_body(q, _):
        qb = q * 16
        flat = idx + qb
        row = flat // 3
        col = flat - row * 3
        v = plsc.load_gather(img_ref, [row, col])
        wsv = plsc.load_gather(ws_ref, [row])
        plsc.store_scatter(img_ref, [row, col], v + (1.0 - wsv) * BG)
        return 0

    lax.fori_loop(0, RPW * 3 // 16, blend_body, 0)

    pltpu.sync_copy(ws_ref, ws_hbm.at[pl.ds(r0, RPW)])
    pltpu.sync_copy(d_ref, d_hbm.at[pl.ds(r0, RPW)])
    pltpu.sync_copy(img_ref, img_hbm.at[pl.ds(r0, RPW)])


@jax.jit
def kernel(sigmas, rgbs, ts, cu_seqlens):
    cu_pad = jnp.concatenate(
        [cu_seqlens, jnp.full((CU_PAD - N - 1,), M, jnp.int32)])
    ts_b = ts.reshape(M // 128, 128, 2).transpose(0, 2, 1)
    rgb_b = rgbs.reshape(M // 128, 128, 3).transpose(0, 2, 1)
    mesh = plsc.VectorSubcoreMesh(core_axis_name="c", subcore_axis_name="s")
    f = pl.kernel(
        _body,
        out_type=(
            jax.ShapeDtypeStruct((M,), jnp.float32),
            jax.ShapeDtypeStruct((N,), jnp.float32),
            jax.ShapeDtypeStruct((N,), jnp.float32),
            jax.ShapeDtypeStruct((N, 3), jnp.float32),
        ),
        mesh=mesh,
        compiler_params=pltpu.CompilerParams(
            needs_layout_passes=False, use_tc_tiling_on_sc=False),
        scratch_types=[
            pltpu.VMEM((CU_PAD,), jnp.int32),
            pltpu.VMEM((2, CH), jnp.float32),
            pltpu.VMEM((2, CH // 128, 2, 128), jnp.float32),
            pltpu.VMEM((2, CH // 128, 3, 128), jnp.float32),
            pltpu.VMEM((2, CH), jnp.float32),
            pltpu.VMEM((RPW,), jnp.float32),
            pltpu.VMEM((RPW,), jnp.float32),
            pltpu.VMEM((RPW, 3), jnp.float32),
            pltpu.SemaphoreType.DMA((2, 3)),
            pltpu.SemaphoreType.DMA((2,)),
        ],
    )
    return f(sigmas, ts_b, rgb_b, cu_pad)
